# Initial kernel scaffold; baseline (speedup 1.0000x reference)
import jax
import jax.numpy as jnp
from jax.experimental import pallas as pl


def kernel(x, edge_index, W1, b1, W2, b2):
    raise NotImplementedError("write your pallas kernel here")


# speedup vs baseline: 12.9138x; 12.9138x over previous
import functools

import jax
import jax.numpy as jnp
from jax import lax
from jax.experimental import pallas as pl
from jax.experimental.pallas import tpu as pltpu
import jax.experimental.pallas.tpu_sc as plsc

N_NODES = 10000
N_EDGES = 320000
D = 128

NC, NS = 2, 16
NW = NC * NS
EPW = N_EDGES // NW
CHUNK = 80
NCHUNKS = EPW // CHUNK
N_PAD = 10240
RPT = N_PAD // NS


def _sc_mesh():
    return plsc.VectorSubcoreMesh(
        core_axis_name="c", subcore_axis_name="s",
        num_cores=NC, num_subcores=NS)


def _deg_body(col_hbm, out_hbm, idx_v, ones_v, zrow_v, acc_sh):
    c = lax.axis_index("c")
    s = lax.axis_index("s")
    wid = c * NS + s
    for j in range(CHUNK // 16):
        ones_v[pl.ds(16 * j, 16)] = jnp.full((16,), 1.0, jnp.float32)
        zrow_v[pl.ds(16 * j, 16)] = jnp.zeros((16,), jnp.float32)
    for k in range(RPT // CHUNK):
        pltpu.sync_copy(zrow_v, acc_sh.at[pl.ds(s * RPT + k * CHUNK, CHUNK)])
    plsc.subcore_barrier()
    base = wid * EPW

    def body(i, carry):
        off = pl.multiple_of(base + i * CHUNK, CHUNK)
        pltpu.sync_copy(col_hbm.at[pl.ds(off, CHUNK)], idx_v)
        pltpu.sync_copy(ones_v, acc_sh.at[idx_v], add=True)
        return carry

    lax.fori_loop(0, NCHUNKS, body, 0)
    plsc.subcore_barrier()
    pltpu.sync_copy(acc_sh.at[pl.ds(s * RPT, RPT)],
                    out_hbm.at[c, pl.ds(s * RPT, RPT)])


def _deg_call(col):
    return pl.kernel(
        _deg_body,
        out_type=jax.ShapeDtypeStruct((NC, N_PAD), jnp.float32),
        mesh=_sc_mesh(),
        scratch_types=[
            pltpu.VMEM((CHUNK,), jnp.int32),
            pltpu.VMEM((CHUNK,), jnp.float32),
            pltpu.VMEM((CHUNK,), jnp.float32),
            pltpu.VMEM_SHARED((N_PAD,), jnp.float32),
        ],
    )(col)


def _agg_body(g_hbm, row_hbm, col_hbm, out_hbm, ridx_v, cidx_v, rows_v,
              acc_sh, sem):
    c = lax.axis_index("c")
    s = lax.axis_index("s")
    wid = c * NS + s
    for r in range(CHUNK):
        for j in range(D // 16):
            rows_v[r, pl.ds(16 * j, 16)] = jnp.zeros((16,), jnp.float32)
    for k in range(RPT // CHUNK):
        pltpu.sync_copy(rows_v, acc_sh.at[pl.ds(s * RPT + k * CHUNK, CHUNK)])
    plsc.subcore_barrier()
    base = wid * EPW

    def body(i, carry):
        off = pl.multiple_of(base + i * CHUNK, CHUNK)
        pltpu.sync_copy(row_hbm.at[pl.ds(off, CHUNK)], ridx_v)
        pltpu.sync_copy(col_hbm.at[pl.ds(off, CHUNK)], cidx_v)
        pltpu.async_copy(g_hbm.at[ridx_v], rows_v, sem).wait()
        pltpu.sync_copy(rows_v, acc_sh.at[cidx_v], add=True)
        return carry

    lax.fori_loop(0, NCHUNKS, body, 0)
    plsc.subcore_barrier()
    pltpu.sync_copy(acc_sh.at[pl.ds(s * RPT, RPT)],
                    out_hbm.at[c, pl.ds(s * RPT, RPT)])


def _agg_call(g, row, col):
    return pl.kernel(
        _agg_body,
        out_type=jax.ShapeDtypeStruct((NC, N_PAD, D), jnp.float32),
        mesh=_sc_mesh(),
        scratch_types=[
            pltpu.VMEM((CHUNK,), jnp.int32),
            pltpu.VMEM((CHUNK,), jnp.int32),
            pltpu.VMEM((CHUNK, D), jnp.float32),
            pltpu.VMEM_SHARED((N_PAD, D), jnp.float32),
            pltpu.SemaphoreType.DMA,
        ],
    )(g, row, col)


BLK = 1000


def _tc_a_body(d0_ref, d1_ref, x_ref, w_ref, dis_ref, g_ref):
    deg = d0_ref[...] + d1_ref[...] + 1.0
    dis = lax.rsqrt(deg)
    h = jnp.dot(x_ref[...], w_ref[...], preferred_element_type=jnp.float32)
    dis_ref[...] = dis
    g_ref[...] = h * dis


_tc_a_call = pl.pallas_call(
    _tc_a_body,
    grid=(N_NODES // BLK,),
    in_specs=[
        pl.BlockSpec((BLK, 1), lambda i: (i, 0)),
        pl.BlockSpec((BLK, 1), lambda i: (i, 0)),
        pl.BlockSpec((BLK, D), lambda i: (i, 0)),
        pl.BlockSpec((D, D), lambda i: (0, 0)),
    ],
    out_specs=[
        pl.BlockSpec((BLK, 1), lambda i: (i, 0)),
        pl.BlockSpec((BLK, D), lambda i: (i, 0)),
    ],
    out_shape=[
        jax.ShapeDtypeStruct((N_NODES, 1), jnp.float32),
        jax.ShapeDtypeStruct((N_NODES, D), jnp.float32),
    ],
)


def _tc_b_body(a0_ref, a1_ref, g1_ref, dis_ref, b_ref, w_ref, g2_ref):
    t = (a0_ref[...] + a1_ref[...] + g1_ref[...]) * dis_ref[...] + b_ref[...]
    a = jnp.maximum(t, 0.0)
    h2 = jnp.dot(a, w_ref[...], preferred_element_type=jnp.float32)
    g2_ref[...] = h2 * dis_ref[...]


_tc_b_call = pl.pallas_call(
    _tc_b_body,
    grid=(N_NODES // BLK,),
    in_specs=[
        pl.BlockSpec((BLK, D), lambda i: (i, 0)),
        pl.BlockSpec((BLK, D), lambda i: (i, 0)),
        pl.BlockSpec((BLK, D), lambda i: (i, 0)),
        pl.BlockSpec((BLK, 1), lambda i: (i, 0)),
        pl.BlockSpec((1, D), lambda i: (0, 0)),
        pl.BlockSpec((D, D), lambda i: (0, 0)),
    ],
    out_specs=pl.BlockSpec((BLK, D), lambda i: (i, 0)),
    out_shape=jax.ShapeDtypeStruct((N_NODES, D), jnp.float32),
)


def _tc_c_body(a0_ref, a1_ref, g2_ref, dis_ref, b_ref, out_ref):
    out_ref[...] = ((a0_ref[...] + a1_ref[...] + g2_ref[...]) * dis_ref[...]
                    + b_ref[...])


_tc_c_call = pl.pallas_call(
    _tc_c_body,
    grid=(N_NODES // BLK,),
    in_specs=[
        pl.BlockSpec((BLK, D), lambda i: (i, 0)),
        pl.BlockSpec((BLK, D), lambda i: (i, 0)),
        pl.BlockSpec((BLK, D), lambda i: (i, 0)),
        pl.BlockSpec((BLK, 1), lambda i: (i, 0)),
        pl.BlockSpec((1, D), lambda i: (0, 0)),
    ],
    out_specs=pl.BlockSpec((BLK, D), lambda i: (i, 0)),
    out_shape=jax.ShapeDtypeStruct((N_NODES, D), jnp.float32),
)


def kernel(x, edge_index, W1, b1, W2, b2):
    row = edge_index[0].astype(jnp.int32)
    col = edge_index[1].astype(jnp.int32)

    degp = _deg_call(col)
    d0 = degp[0, :N_NODES, None]
    d1 = degp[1, :N_NODES, None]

    dis, g1 = _tc_a_call(d0, d1, x, W1)
    acc1 = _agg_call(g1, row, col)
    g2 = _tc_b_call(acc1[0, :N_NODES], acc1[1, :N_NODES], g1, dis,
                    b1.reshape(1, D), W2)
    acc2 = _agg_call(g2, row, col)
    out = _tc_c_call(acc2[0, :N_NODES], acc2[1, :N_NODES], g2, dis,
                     b2.reshape(1, D))
    return out


# speedup vs baseline: 21.8382x; 1.6911x over previous
import functools

import jax
import jax.numpy as jnp
from jax import lax
from jax.experimental import pallas as pl
from jax.experimental.pallas import tpu as pltpu
import jax.experimental.pallas.tpu_sc as plsc

N_NODES = 10000
N_EDGES = 320000
D = 128

NC, NS = 2, 16
NW = NC * NS
EPW = N_EDGES // NW
CHUNK = 80
NCHUNKS = EPW // CHUNK
N_PAD = 10240
RPT = N_PAD // NS


def _sc_mesh():
    return plsc.VectorSubcoreMesh(
        core_axis_name="c", subcore_axis_name="s",
        num_cores=NC, num_subcores=NS)


def _deg_body(col_hbm, out_hbm, cidx_v, ones_v, acc_sh, isem):
    c = lax.axis_index("c")
    s = lax.axis_index("s")
    wid = c * NS + s
    for j in range(CHUNK // 16):
        ones_v[pl.ds(16 * j, 16)] = jnp.full((16,), 1.0, jnp.float32)
        ones_v[pl.ds(CHUNK + 16 * j, 16)] = jnp.zeros((16,), jnp.float32)
    for k in range(RPT // CHUNK):
        pltpu.sync_copy(ones_v.at[pl.ds(CHUNK, CHUNK)],
                        acc_sh.at[pl.ds(s * RPT + k * CHUNK, CHUNK)])
    plsc.subcore_barrier()

    base = wid * NCHUNKS

    def fire_idx(i, b):
        pltpu.async_copy(col_hbm.at[pl.ds(base + i, 1)],
                         cidx_v.at[pl.ds(b, 1)], isem)

    def wait_idx(i, b):
        pltpu.make_async_copy(col_hbm.at[pl.ds(base + i, 1)],
                              cidx_v.at[pl.ds(b, 1)], isem).wait()

    fire_idx(0, 0)

    def group(gidx, carry):
        i0 = gidx * 2
        wait_idx(i0, 0)
        fire_idx(i0 + 1, 1)
        pltpu.sync_copy(ones_v.at[pl.ds(0, CHUNK)],
                        acc_sh.at[cidx_v.at[0]], add=True)
        wait_idx(i0 + 1, 1)
        fire_idx(i0 + 2, 0)
        pltpu.sync_copy(ones_v.at[pl.ds(0, CHUNK)],
                        acc_sh.at[cidx_v.at[1]], add=True)
        return carry

    lax.fori_loop(0, (NCHUNKS - 1) // 2, group, 0)
    wait_idx(NCHUNKS - 1, 0)
    pltpu.sync_copy(ones_v.at[pl.ds(0, CHUNK)],
                    acc_sh.at[cidx_v.at[0]], add=True)
    plsc.subcore_barrier()
    pltpu.sync_copy(acc_sh.at[pl.ds(s * RPT, RPT)],
                    out_hbm.at[c, pl.ds(s * RPT, RPT)])


def _deg_call(col_flat):
    return pl.kernel(
        _deg_body,
        out_type=jax.ShapeDtypeStruct((NC, N_PAD), jnp.float32),
        mesh=_sc_mesh(),
        scratch_types=[
            pltpu.VMEM((2, CHUNK), jnp.int32),
            pltpu.VMEM((2 * CHUNK,), jnp.float32),
            pltpu.VMEM_SHARED((N_PAD,), jnp.float32),
            pltpu.SemaphoreType.DMA,
        ],
    )(col_flat)


def _agg_body(g_hbm, row_hbm, col_hbm, out_hbm, ridx_v, cidx_v, rows_v,
              acc_sh, gsem, irsem, icsem):
    c = lax.axis_index("c")
    s = lax.axis_index("s")
    wid = c * NS + s
    for r in range(CHUNK):
        for j in range(D // 16):
            rows_v[0, r, pl.ds(16 * j, 16)] = jnp.zeros((16,), jnp.float32)
    for k in range(RPT // CHUNK):
        pltpu.sync_copy(rows_v.at[0],
                        acc_sh.at[pl.ds(s * RPT + k * CHUNK, CHUNK)])
    plsc.subcore_barrier()
    base = wid * NCHUNKS

    def fire_ridx(i, b):
        pltpu.async_copy(row_hbm.at[pl.ds(base + i, 1)],
                         ridx_v.at[pl.ds(b, 1)], irsem)

    def wait_ridx(i, b):
        pltpu.make_async_copy(row_hbm.at[pl.ds(base + i, 1)],
                              ridx_v.at[pl.ds(b, 1)], irsem).wait()

    def fire_cidx(i, b):
        pltpu.async_copy(col_hbm.at[pl.ds(base + i, 1)],
                         cidx_v.at[pl.ds(b, 1)], icsem)

    def wait_cidx(i, b):
        pltpu.make_async_copy(col_hbm.at[pl.ds(base + i, 1)],
                              cidx_v.at[pl.ds(b, 1)], icsem).wait()

    def fire_gather(b):
        pltpu.async_copy(g_hbm.at[ridx_v.at[b]], rows_v.at[b], gsem)

    def wait_gather(b):
        pltpu.make_async_copy(g_hbm.at[ridx_v.at[b]], rows_v.at[b],
                              gsem).wait()

    def scat(b):
        pltpu.sync_copy(rows_v.at[b], acc_sh.at[cidx_v.at[b]], add=True)

    def steps(i, b, fire_r2, fire_g1, fire_c1):
        wait_gather(b)
        if fire_r2:
            fire_ridx(i + 2, b)
        if fire_g1:
            wait_ridx(i + 1, 1 - b)
            fire_gather(1 - b)
            fire_cidx(i + 1, 1 - b)
        wait_cidx(i, b)
        scat(b)

    fire_ridx(0, 0)
    wait_ridx(0, 0)
    fire_gather(0)
    fire_ridx(1, 1)
    fire_cidx(0, 0)

    def group(gidx, carry):
        i0 = gidx * 2

        def dyn_steps(i, b):
            wait_gather(b)
            fire_ridx(i + 2, b)
            wait_ridx(i + 1, 1 - b)
            fire_gather(1 - b)
            fire_cidx(i + 1, 1 - b)
            wait_cidx(i, b)
            scat(b)

        dyn_steps(i0, 0)
        dyn_steps(i0 + 1, 1)
        return carry

    lax.fori_loop(0, (NCHUNKS - 3) // 2, group, 0)
    steps(NCHUNKS - 3, 0, True, True, True)
    steps(NCHUNKS - 2, 1, False, True, True)
    steps(NCHUNKS - 1, 0, False, False, False)
    plsc.subcore_barrier()
    pltpu.sync_copy(acc_sh.at[pl.ds(s * RPT, RPT)],
                    out_hbm.at[c, pl.ds(s * RPT, RPT)])


def _agg_call(g, row_flat, col_flat):
    return pl.kernel(
        _agg_body,
        out_type=jax.ShapeDtypeStruct((NC, N_PAD, D), jnp.float32),
        mesh=_sc_mesh(),
        scratch_types=[
            pltpu.VMEM((2, CHUNK), jnp.int32),
            pltpu.VMEM((2, CHUNK), jnp.int32),
            pltpu.VMEM((2, CHUNK, D), jnp.float32),
            pltpu.VMEM_SHARED((N_PAD, D), jnp.float32),
            pltpu.SemaphoreType.DMA,
            pltpu.SemaphoreType.DMA,
            pltpu.SemaphoreType.DMA,
        ],
    )(g, row_flat, col_flat)


BLK = 1000


def _tc_a_body(d0_ref, d1_ref, x_ref, w_ref, dis_ref, g_ref):
    deg = d0_ref[...] + d1_ref[...] + 1.0
    dis = lax.rsqrt(deg)
    h = jnp.dot(x_ref[...], w_ref[...], preferred_element_type=jnp.float32)
    dis_ref[...] = dis
    g_ref[...] = h * dis


_tc_a_call = pl.pallas_call(
    _tc_a_body,
    grid=(N_NODES // BLK,),
    in_specs=[
        pl.BlockSpec((BLK, 1), lambda i: (i, 0)),
        pl.BlockSpec((BLK, 1), lambda i: (i, 0)),
        pl.BlockSpec((BLK, D), lambda i: (i, 0)),
        pl.BlockSpec((D, D), lambda i: (0, 0)),
    ],
    out_specs=[
        pl.BlockSpec((BLK, 1), lambda i: (i, 0)),
        pl.BlockSpec((BLK, D), lambda i: (i, 0)),
    ],
    out_shape=[
        jax.ShapeDtypeStruct((N_NODES, 1), jnp.float32),
        jax.ShapeDtypeStruct((N_NODES, D), jnp.float32),
    ],
)


def _tc_b_body(a0_ref, a1_ref, g1_ref, dis_ref, b_ref, w_ref, g2_ref):
    t = (a0_ref[...] + a1_ref[...] + g1_ref[...]) * dis_ref[...] + b_ref[...]
    a = jnp.maximum(t, 0.0)
    h2 = jnp.dot(a, w_ref[...], preferred_element_type=jnp.float32)
    g2_ref[...] = h2 * dis_ref[...]


_tc_b_call = pl.pallas_call(
    _tc_b_body,
    grid=(N_NODES // BLK,),
    in_specs=[
        pl.BlockSpec((BLK, D), lambda i: (i, 0)),
        pl.BlockSpec((BLK, D), lambda i: (i, 0)),
        pl.BlockSpec((BLK, D), lambda i: (i, 0)),
        pl.BlockSpec((BLK, 1), lambda i: (i, 0)),
        pl.BlockSpec((1, D), lambda i: (0, 0)),
        pl.BlockSpec((D, D), lambda i: (0, 0)),
    ],
    out_specs=pl.BlockSpec((BLK, D), lambda i: (i, 0)),
    out_shape=jax.ShapeDtypeStruct((N_NODES, D), jnp.float32),
)


def _tc_c_body(a0_ref, a1_ref, g2_ref, dis_ref, b_ref, out_ref):
    out_ref[...] = ((a0_ref[...] + a1_ref[...] + g2_ref[...]) * dis_ref[...]
                    + b_ref[...])


_tc_c_call = pl.pallas_call(
    _tc_c_body,
    grid=(N_NODES // BLK,),
    in_specs=[
        pl.BlockSpec((BLK, D), lambda i: (i, 0)),
        pl.BlockSpec((BLK, D), lambda i: (i, 0)),
        pl.BlockSpec((BLK, D), lambda i: (i, 0)),
        pl.BlockSpec((BLK, 1), lambda i: (i, 0)),
        pl.BlockSpec((1, D), lambda i: (0, 0)),
    ],
    out_specs=pl.BlockSpec((BLK, D), lambda i: (i, 0)),
    out_shape=jax.ShapeDtypeStruct((N_NODES, D), jnp.float32),
)


def kernel(x, edge_index, W1, b1, W2, b2):
    row = edge_index[0].astype(jnp.int32)
    col = edge_index[1].astype(jnp.int32)
    row2 = row.reshape(NW * NCHUNKS, CHUNK)
    col2 = col.reshape(NW * NCHUNKS, CHUNK)

    degp = _deg_call(col2)
    d0 = degp[0, :N_NODES, None]
    d1 = degp[1, :N_NODES, None]

    dis, g1 = _tc_a_call(d0, d1, x, W1)
    acc1 = _agg_call(g1, row2, col2)
    g2 = _tc_b_call(acc1[0, :N_NODES], acc1[1, :N_NODES], g1, dis,
                    b1.reshape(1, D), W2)
    acc2 = _agg_call(g2, row2, col2)
    out = _tc_c_call(acc2[0, :N_NODES], acc2[1, :N_NODES], g2, dis,
                     b2.reshape(1, D))
    return out


# speedup vs baseline: 23.5575x; 1.0787x over previous
import functools

import jax
import jax.numpy as jnp
from jax import lax
from jax.experimental import pallas as pl
from jax.experimental.pallas import tpu as pltpu
import jax.experimental.pallas.tpu_sc as plsc

N_NODES = 10000
N_EDGES = 320000
D = 128

NC, NS = 2, 16
NW = NC * NS
EPW = N_EDGES // NW
CHUNK = 80
NCHUNKS = EPW // CHUNK
N_PAD = 10240
ACHUNK = 40
ANCH = EPW // ACHUNK
RPT = N_PAD // NS


def _sc_mesh():
    return plsc.VectorSubcoreMesh(
        core_axis_name="c", subcore_axis_name="s",
        num_cores=NC, num_subcores=NS)


def _deg_body(col_hbm, out_hbm, cidx_v, ones_v, acc_sh, isem):
    c = lax.axis_index("c")
    s = lax.axis_index("s")
    wid = c * NS + s
    for j in range(CHUNK // 16):
        ones_v[pl.ds(16 * j, 16)] = jnp.full((16,), 1.0, jnp.float32)
        ones_v[pl.ds(CHUNK + 16 * j, 16)] = jnp.zeros((16,), jnp.float32)
    for k in range(RPT // CHUNK):
        pltpu.sync_copy(ones_v.at[pl.ds(CHUNK, CHUNK)],
                        acc_sh.at[pl.ds(s * RPT + k * CHUNK, CHUNK)])
    plsc.subcore_barrier()

    base = wid * NCHUNKS

    def fire_idx(i, b):
        pltpu.async_copy(col_hbm.at[pl.ds(base + i, 1)],
                         cidx_v.at[pl.ds(b, 1)], isem)

    def wait_idx(i, b):
        pltpu.make_async_copy(col_hbm.at[pl.ds(base + i, 1)],
                              cidx_v.at[pl.ds(b, 1)], isem).wait()

    fire_idx(0, 0)

    def group(gidx, carry):
        i0 = gidx * 2
        wait_idx(i0, 0)
        fire_idx(i0 + 1, 1)
        pltpu.sync_copy(ones_v.at[pl.ds(0, CHUNK)],
                        acc_sh.at[cidx_v.at[0]], add=True)
        wait_idx(i0 + 1, 1)
        fire_idx(i0 + 2, 0)
        pltpu.sync_copy(ones_v.at[pl.ds(0, CHUNK)],
                        acc_sh.at[cidx_v.at[1]], add=True)
        return carry

    lax.fori_loop(0, (NCHUNKS - 1) // 2, group, 0)
    wait_idx(NCHUNKS - 1, 0)
    pltpu.sync_copy(ones_v.at[pl.ds(0, CHUNK)],
                    acc_sh.at[cidx_v.at[0]], add=True)
    plsc.subcore_barrier()
    pltpu.sync_copy(acc_sh.at[pl.ds(s * RPT, RPT)],
                    out_hbm.at[c, pl.ds(s * RPT, RPT)])


def _deg_call(col_flat):
    return pl.kernel(
        _deg_body,
        out_type=jax.ShapeDtypeStruct((NC, N_PAD), jnp.float32),
        mesh=_sc_mesh(),
        scratch_types=[
            pltpu.VMEM((2, CHUNK), jnp.int32),
            pltpu.VMEM((2 * CHUNK,), jnp.float32),
            pltpu.VMEM_SHARED((N_PAD,), jnp.float32),
            pltpu.SemaphoreType.DMA,
        ],
    )(col_flat)


def _agg_body(g_hbm, row_hbm, col_hbm, out_hbm, ridx_v, cidx_v, rows_v,
              acc_sh, gsem, ssem, irsem, icsem):
    c = lax.axis_index("c")
    s = lax.axis_index("s")
    wid = c * NS + s
    for r in range(ACHUNK):
        for j in range(D // 16):
            rows_v[0, r, pl.ds(16 * j, 16)] = jnp.zeros((16,), jnp.float32)
    for k in range(RPT // ACHUNK):
        pltpu.sync_copy(rows_v.at[0],
                        acc_sh.at[pl.ds(s * RPT + k * ACHUNK, ACHUNK)])
    plsc.subcore_barrier()
    base = wid * ANCH

    def fire_ridx(i, b):
        pltpu.async_copy(row_hbm.at[pl.ds(base + i, 1)],
                         ridx_v.at[pl.ds(b, 1)], irsem)

    def wait_ridx(i, b):
        pltpu.make_async_copy(row_hbm.at[pl.ds(base + i, 1)],
                              ridx_v.at[pl.ds(b, 1)], irsem).wait()

    def fire_cidx(i, b):
        pltpu.async_copy(col_hbm.at[pl.ds(base + i, 1)],
                         cidx_v.at[pl.ds(b, 1)], icsem)

    def wait_cidx(i, b):
        pltpu.make_async_copy(col_hbm.at[pl.ds(base + i, 1)],
                              cidx_v.at[pl.ds(b, 1)], icsem).wait()

    def fire_gather(b):
        pltpu.async_copy(g_hbm.at[ridx_v.at[b]], rows_v.at[b], gsem)

    def wait_gather(b):
        pltpu.make_async_copy(g_hbm.at[ridx_v.at[b]], rows_v.at[b],
                              gsem).wait()

    def fire_scat(b):
        pltpu.async_copy(rows_v.at[b], acc_sh.at[cidx_v.at[b]], ssem,
                         add=True)

    def wait_scat(b):
        pltpu.make_async_copy(rows_v.at[b], acc_sh.at[cidx_v.at[b]],
                              ssem).wait()

    fire_ridx(0, 0)
    fire_ridx(1, 1)
    fire_ridx(2, 2)
    fire_cidx(0, 0)
    fire_cidx(1, 1)
    wait_ridx(0, 0)
    fire_gather(0)
    wait_ridx(1, 1)
    fire_gather(1)

    wait_gather(0); wait_cidx(0, 0); fire_scat(0)
    fire_ridx(3, 3); wait_ridx(2, 2); fire_gather(2); fire_cidx(2, 2)
    wait_gather(1); wait_cidx(1, 1); fire_scat(1)
    fire_ridx(4, 0); wait_ridx(3, 3); fire_gather(3); fire_cidx(3, 3)
    for i, b in ((2, 2), (3, 3)):
        wait_gather(b); wait_cidx(i, b); fire_scat(b)
        wait_scat((i - 2) % 4)
        fire_ridx(i + 3, (i + 3) % 4)
        wait_ridx(i + 2, (i + 2) % 4)
        fire_gather((i + 2) % 4)
        fire_cidx(i + 2, (i + 2) % 4)

    def group(gidx, carry):
        i0 = gidx * 4 + 4
        for b in range(4):
            i = i0 + b
            wait_gather(b); wait_cidx(i, b); fire_scat(b)
            wait_scat((b + 2) % 4)
            fire_ridx(i + 3, (b + 3) % 4)
            wait_ridx(i + 2, (b + 2) % 4)
            fire_gather((b + 2) % 4)
            fire_cidx(i + 2, (b + 2) % 4)
        return carry

    lax.fori_loop(0, (ANCH - 10) // 4, group, 0)
    n6 = ANCH - 6
    wait_gather(0); wait_cidx(n6, 0); fire_scat(0); wait_scat(2)
    fire_ridx(n6 + 3, 3); wait_ridx(n6 + 2, 2); fire_gather(2)
    fire_cidx(n6 + 2, 2)
    wait_gather(1); wait_cidx(n6 + 1, 1); fire_scat(1); wait_scat(3)
    fire_ridx(n6 + 4, 0); wait_ridx(n6 + 3, 3); fire_gather(3)
    fire_cidx(n6 + 3, 3)
    wait_gather(2); wait_cidx(n6 + 2, 2); fire_scat(2); wait_scat(0)
    fire_ridx(n6 + 5, 1); wait_ridx(n6 + 4, 0); fire_gather(0)
    fire_cidx(n6 + 4, 0)
    wait_gather(3); wait_cidx(n6 + 3, 3); fire_scat(3); wait_scat(1)
    wait_ridx(n6 + 5, 1); fire_gather(1); fire_cidx(n6 + 5, 1)
    wait_gather(0); wait_cidx(n6 + 4, 0); fire_scat(0); wait_scat(2)
    wait_gather(1); wait_cidx(n6 + 5, 1); fire_scat(1); wait_scat(3)
    wait_scat(0)
    wait_scat(1)
    plsc.subcore_barrier()
    pltpu.sync_copy(acc_sh.at[pl.ds(s * RPT, RPT)],
                    out_hbm.at[c, pl.ds(s * RPT, RPT)])


def _agg_call(g, row_flat, col_flat):
    return pl.kernel(
        _agg_body,
        out_type=jax.ShapeDtypeStruct((NC, N_PAD, D), jnp.float32),
        mesh=_sc_mesh(),
        scratch_types=[
            pltpu.VMEM((4, ACHUNK), jnp.int32),
            pltpu.VMEM((4, ACHUNK), jnp.int32),
            pltpu.VMEM((4, ACHUNK, D), jnp.float32),
            pltpu.VMEM_SHARED((N_PAD, D), jnp.float32),
            pltpu.SemaphoreType.DMA,
            pltpu.SemaphoreType.DMA,
            pltpu.SemaphoreType.DMA,
            pltpu.SemaphoreType.DMA,
        ],
    )(g, row_flat, col_flat)


BLK = 1000


def _tc_a_body(d0_ref, d1_ref, x_ref, w_ref, dis_ref, g_ref):
    deg = d0_ref[...] + d1_ref[...] + 1.0
    dis = lax.rsqrt(deg)
    h = jnp.dot(x_ref[...], w_ref[...], preferred_element_type=jnp.float32)
    dis_ref[...] = dis
    g_ref[...] = h * dis


_tc_a_call = pl.pallas_call(
    _tc_a_body,
    grid=(N_NODES // BLK,),
    in_specs=[
        pl.BlockSpec((BLK, 1), lambda i: (i, 0)),
        pl.BlockSpec((BLK, 1), lambda i: (i, 0)),
        pl.BlockSpec((BLK, D), lambda i: (i, 0)),
        pl.BlockSpec((D, D), lambda i: (0, 0)),
    ],
    out_specs=[
        pl.BlockSpec((BLK, 1), lambda i: (i, 0)),
        pl.BlockSpec((BLK, D), lambda i: (i, 0)),
    ],
    out_shape=[
        jax.ShapeDtypeStruct((N_NODES, 1), jnp.float32),
        jax.ShapeDtypeStruct((N_NODES, D), jnp.float32),
    ],
)


def _tc_b_body(a0_ref, a1_ref, g1_ref, dis_ref, b_ref, w_ref, g2_ref):
    t = (a0_ref[...] + a1_ref[...] + g1_ref[...]) * dis_ref[...] + b_ref[...]
    a = jnp.maximum(t, 0.0)
    h2 = jnp.dot(a, w_ref[...], preferred_element_type=jnp.float32)
    g2_ref[...] = h2 * dis_ref[...]


_tc_b_call = pl.pallas_call(
    _tc_b_body,
    grid=(N_NODES // BLK,),
    in_specs=[
        pl.BlockSpec((BLK, D), lambda i: (i, 0)),
        pl.BlockSpec((BLK, D), lambda i: (i, 0)),
        pl.BlockSpec((BLK, D), lambda i: (i, 0)),
        pl.BlockSpec((BLK, 1), lambda i: (i, 0)),
        pl.BlockSpec((1, D), lambda i: (0, 0)),
        pl.BlockSpec((D, D), lambda i: (0, 0)),
    ],
    out_specs=pl.BlockSpec((BLK, D), lambda i: (i, 0)),
    out_shape=jax.ShapeDtypeStruct((N_NODES, D), jnp.float32),
)


def _tc_c_body(a0_ref, a1_ref, g2_ref, dis_ref, b_ref, out_ref):
    out_ref[...] = ((a0_ref[...] + a1_ref[...] + g2_ref[...]) * dis_ref[...]
                    + b_ref[...])


_tc_c_call = pl.pallas_call(
    _tc_c_body,
    grid=(N_NODES // BLK,),
    in_specs=[
        pl.BlockSpec((BLK, D), lambda i: (i, 0)),
        pl.BlockSpec((BLK, D), lambda i: (i, 0)),
        pl.BlockSpec((BLK, D), lambda i: (i, 0)),
        pl.BlockSpec((BLK, 1), lambda i: (i, 0)),
        pl.BlockSpec((1, D), lambda i: (0, 0)),
    ],
    out_specs=pl.BlockSpec((BLK, D), lambda i: (i, 0)),
    out_shape=jax.ShapeDtypeStruct((N_NODES, D), jnp.float32),
)


def kernel(x, edge_index, W1, b1, W2, b2):
    row = edge_index[0].astype(jnp.int32)
    col = edge_index[1].astype(jnp.int32)
    row2 = row.reshape(NW * NCHUNKS, CHUNK)
    col2 = col.reshape(NW * NCHUNKS, CHUNK)
    row2a = row.reshape(NW * ANCH, ACHUNK)
    col2a = col.reshape(NW * ANCH, ACHUNK)

    degp = _deg_call(col2)
    d0 = degp[0, :N_NODES, None]
    d1 = degp[1, :N_NODES, None]

    dis, g1 = _tc_a_call(d0, d1, x, W1)
    acc1 = _agg_call(g1, row2a, col2a)
    g2 = _tc_b_call(acc1[0, :N_NODES], acc1[1, :N_NODES], g1, dis,
                    b1.reshape(1, D), W2)
    acc2 = _agg_call(g2, row2a, col2a)
    out = _tc_c_call(acc2[0, :N_NODES], acc2[1, :N_NODES], g2, dis,
                     b2.reshape(1, D))
    return out


# speedup vs baseline: 23.6620x; 1.0044x over previous
import functools

import jax
import jax.numpy as jnp
from jax import lax
from jax.experimental import pallas as pl
from jax.experimental.pallas import tpu as pltpu
import jax.experimental.pallas.tpu_sc as plsc

N_NODES = 10000
N_EDGES = 320000
D = 128

NC, NS = 2, 16
NW = NC * NS
EPW = N_EDGES // NW
CHUNK = 80
NCHUNKS = EPW // CHUNK
N_PAD = 10240
ACHUNK = 40
ANCH = EPW // ACHUNK
RPT = N_PAD // NS


def _sc_mesh():
    return plsc.VectorSubcoreMesh(
        core_axis_name="c", subcore_axis_name="s",
        num_cores=NC, num_subcores=NS)


def _deg_body(col_hbm, out_hbm, cidx_v, ones_v, acc_sh, isem):
    c = lax.axis_index("c")
    s = lax.axis_index("s")
    wid = c * NS + s
    for j in range(CHUNK // 16):
        ones_v[pl.ds(16 * j, 16)] = jnp.full((16,), 1.0, jnp.float32)
        ones_v[pl.ds(CHUNK + 16 * j, 16)] = jnp.zeros((16,), jnp.float32)
    for k in range(RPT // CHUNK):
        pltpu.sync_copy(ones_v.at[pl.ds(CHUNK, CHUNK)],
                        acc_sh.at[pl.ds(s * RPT + k * CHUNK, CHUNK)])
    plsc.subcore_barrier()

    base = wid * NCHUNKS

    def fire_idx(i, b):
        pltpu.async_copy(col_hbm.at[pl.ds(base + i, 1)],
                         cidx_v.at[pl.ds(b, 1)], isem)

    def wait_idx(i, b):
        pltpu.make_async_copy(col_hbm.at[pl.ds(base + i, 1)],
                              cidx_v.at[pl.ds(b, 1)], isem).wait()

    fire_idx(0, 0)

    def group(gidx, carry):
        i0 = gidx * 2
        wait_idx(i0, 0)
        fire_idx(i0 + 1, 1)
        pltpu.sync_copy(ones_v.at[pl.ds(0, CHUNK)],
                        acc_sh.at[cidx_v.at[0]], add=True)
        wait_idx(i0 + 1, 1)
        fire_idx(i0 + 2, 0)
        pltpu.sync_copy(ones_v.at[pl.ds(0, CHUNK)],
                        acc_sh.at[cidx_v.at[1]], add=True)
        return carry

    lax.fori_loop(0, (NCHUNKS - 1) // 2, group, 0)
    wait_idx(NCHUNKS - 1, 0)
    pltpu.sync_copy(ones_v.at[pl.ds(0, CHUNK)],
                    acc_sh.at[cidx_v.at[0]], add=True)
    plsc.subcore_barrier()
    pltpu.sync_copy(acc_sh.at[pl.ds(s * RPT, RPT)],
                    out_hbm.at[c, pl.ds(s * RPT, RPT)])


def _deg_call(col_flat):
    return pl.kernel(
        _deg_body,
        out_type=jax.ShapeDtypeStruct((NC, N_PAD), jnp.float32),
        mesh=_sc_mesh(),
        scratch_types=[
            pltpu.VMEM((2, CHUNK), jnp.int32),
            pltpu.VMEM((2 * CHUNK,), jnp.float32),
            pltpu.VMEM_SHARED((N_PAD,), jnp.float32),
            pltpu.SemaphoreType.DMA,
        ],
    )(col_flat)


def _agg_body(g_hbm, row_hbm, col_hbm, out_hbm, ridx_v, cidx_v, rows_v,
              acc_sh, gsem, ssem, irsem, icsem):
    c = lax.axis_index("c")
    s = lax.axis_index("s")
    wid = c * NS + s
    base = wid * ANCH

    def fire_ridx(i, b):
        pltpu.async_copy(row_hbm.at[pl.ds(base + i, 1)],
                         ridx_v.at[pl.ds(b, 1)], irsem)

    def wait_ridx(i, b):
        pltpu.make_async_copy(row_hbm.at[pl.ds(base + i, 1)],
                              ridx_v.at[pl.ds(b, 1)], irsem).wait()

    def fire_cidx(i, b):
        pltpu.async_copy(col_hbm.at[pl.ds(base + i, 1)],
                         cidx_v.at[pl.ds(b, 1)], icsem)

    def wait_cidx(i, b):
        pltpu.make_async_copy(col_hbm.at[pl.ds(base + i, 1)],
                              cidx_v.at[pl.ds(b, 1)], icsem).wait()

    def fire_gather(b):
        pltpu.async_copy(g_hbm.at[ridx_v.at[b]], rows_v.at[b], gsem)

    def wait_gather(b):
        pltpu.make_async_copy(g_hbm.at[ridx_v.at[b]], rows_v.at[b],
                              gsem).wait()

    def fire_scat(b):
        pltpu.async_copy(rows_v.at[b], acc_sh.at[cidx_v.at[b]], ssem,
                         add=True)

    def wait_scat(b):
        pltpu.make_async_copy(rows_v.at[b], acc_sh.at[cidx_v.at[b]],
                              ssem).wait()

    fire_ridx(0, 0)
    fire_ridx(1, 1)
    fire_ridx(2, 2)
    fire_cidx(0, 0)
    fire_cidx(1, 1)
    wait_ridx(0, 0)
    fire_gather(0)
    wait_ridx(1, 1)
    fire_gather(1)
    for r in range(ACHUNK):
        for j in range(D // 16):
            rows_v[3, r, pl.ds(16 * j, 16)] = jnp.zeros((16,), jnp.float32)
    for k in range(RPT // ACHUNK):
        pltpu.sync_copy(rows_v.at[3],
                        acc_sh.at[pl.ds(s * RPT + k * ACHUNK, ACHUNK)])
    plsc.subcore_barrier()

    wait_gather(0); wait_cidx(0, 0); fire_scat(0)
    fire_ridx(3, 3); wait_ridx(2, 2); fire_gather(2); fire_cidx(2, 2)
    wait_gather(1); wait_cidx(1, 1); fire_scat(1)
    fire_ridx(4, 0); wait_ridx(3, 3); fire_gather(3); fire_cidx(3, 3)
    for i, b in ((2, 2), (3, 3)):
        wait_gather(b); wait_cidx(i, b); fire_scat(b)
        wait_scat((i - 2) % 4)
        fire_ridx(i + 3, (i + 3) % 4)
        wait_ridx(i + 2, (i + 2) % 4)
        fire_gather((i + 2) % 4)
        fire_cidx(i + 2, (i + 2) % 4)

    def group(gidx, carry):
        i0 = gidx * 4 + 4
        for b in range(4):
            i = i0 + b
            wait_gather(b); wait_cidx(i, b); fire_scat(b)
            wait_scat((b + 2) % 4)
            fire_ridx(i + 3, (b + 3) % 4)
            wait_ridx(i + 2, (b + 2) % 4)
            fire_gather((b + 2) % 4)
            fire_cidx(i + 2, (b + 2) % 4)
        return carry

    lax.fori_loop(0, (ANCH - 10) // 4, group, 0)
    n6 = ANCH - 6
    wait_gather(0); wait_cidx(n6, 0); fire_scat(0); wait_scat(2)
    fire_ridx(n6 + 3, 3); wait_ridx(n6 + 2, 2); fire_gather(2)
    fire_cidx(n6 + 2, 2)
    wait_gather(1); wait_cidx(n6 + 1, 1); fire_scat(1); wait_scat(3)
    fire_ridx(n6 + 4, 0); wait_ridx(n6 + 3, 3); fire_gather(3)
    fire_cidx(n6 + 3, 3)
    wait_gather(2); wait_cidx(n6 + 2, 2); fire_scat(2); wait_scat(0)
    fire_ridx(n6 + 5, 1); wait_ridx(n6 + 4, 0); fire_gather(0)
    fire_cidx(n6 + 4, 0)
    wait_gather(3); wait_cidx(n6 + 3, 3); fire_scat(3); wait_scat(1)
    wait_ridx(n6 + 5, 1); fire_gather(1); fire_cidx(n6 + 5, 1)
    wait_gather(0); wait_cidx(n6 + 4, 0); fire_scat(0); wait_scat(2)
    wait_gather(1); wait_cidx(n6 + 5, 1); fire_scat(1); wait_scat(3)
    wait_scat(0)
    wait_scat(1)
    plsc.subcore_barrier()
    pltpu.sync_copy(acc_sh.at[pl.ds(s * RPT, RPT)],
                    out_hbm.at[c, pl.ds(s * RPT, RPT)])


def _agg_call(g, row_flat, col_flat):
    return pl.kernel(
        _agg_body,
        out_type=jax.ShapeDtypeStruct((NC, N_PAD, D), jnp.float32),
        mesh=_sc_mesh(),
        scratch_types=[
            pltpu.VMEM((4, ACHUNK), jnp.int32),
            pltpu.VMEM((4, ACHUNK), jnp.int32),
            pltpu.VMEM((4, ACHUNK, D), jnp.float32),
            pltpu.VMEM_SHARED((N_PAD, D), jnp.float32),
            pltpu.SemaphoreType.DMA,
            pltpu.SemaphoreType.DMA,
            pltpu.SemaphoreType.DMA,
            pltpu.SemaphoreType.DMA,
        ],
    )(g, row_flat, col_flat)


BLK = 1000


def _tc_a_body(d0_ref, d1_ref, x_ref, w_ref, dis_ref, g_ref):
    deg = d0_ref[...] + d1_ref[...] + 1.0
    dis = lax.rsqrt(deg)
    h = jnp.dot(x_ref[...], w_ref[...], preferred_element_type=jnp.float32)
    dis_ref[...] = dis
    g_ref[...] = h * dis


_tc_a_call = pl.pallas_call(
    _tc_a_body,
    grid=(N_NODES // BLK,),
    in_specs=[
        pl.BlockSpec((BLK, 1), lambda i: (i, 0)),
        pl.BlockSpec((BLK, 1), lambda i: (i, 0)),
        pl.BlockSpec((BLK, D), lambda i: (i, 0)),
        pl.BlockSpec((D, D), lambda i: (0, 0)),
    ],
    out_specs=[
        pl.BlockSpec((BLK, 1), lambda i: (i, 0)),
        pl.BlockSpec((BLK, D), lambda i: (i, 0)),
    ],
    out_shape=[
        jax.ShapeDtypeStruct((N_NODES, 1), jnp.float32),
        jax.ShapeDtypeStruct((N_NODES, D), jnp.float32),
    ],
)


def _tc_b_body(a0_ref, a1_ref, g1_ref, dis_ref, b_ref, w_ref, g2_ref):
    t = (a0_ref[...] + a1_ref[...] + g1_ref[...]) * dis_ref[...] + b_ref[...]
    a = jnp.maximum(t, 0.0)
    h2 = jnp.dot(a, w_ref[...], preferred_element_type=jnp.float32)
    g2_ref[...] = h2 * dis_ref[...]


_tc_b_call = pl.pallas_call(
    _tc_b_body,
    grid=(N_NODES // BLK,),
    in_specs=[
        pl.BlockSpec((BLK, D), lambda i: (i, 0)),
        pl.BlockSpec((BLK, D), lambda i: (i, 0)),
        pl.BlockSpec((BLK, D), lambda i: (i, 0)),
        pl.BlockSpec((BLK, 1), lambda i: (i, 0)),
        pl.BlockSpec((1, D), lambda i: (0, 0)),
        pl.BlockSpec((D, D), lambda i: (0, 0)),
    ],
    out_specs=pl.BlockSpec((BLK, D), lambda i: (i, 0)),
    out_shape=jax.ShapeDtypeStruct((N_NODES, D), jnp.float32),
)


def _tc_c_body(a0_ref, a1_ref, g2_ref, dis_ref, b_ref, out_ref):
    out_ref[...] = ((a0_ref[...] + a1_ref[...] + g2_ref[...]) * dis_ref[...]
                    + b_ref[...])


_tc_c_call = pl.pallas_call(
    _tc_c_body,
    grid=(N_NODES // BLK,),
    in_specs=[
        pl.BlockSpec((BLK, D), lambda i: (i, 0)),
        pl.BlockSpec((BLK, D), lambda i: (i, 0)),
        pl.BlockSpec((BLK, D), lambda i: (i, 0)),
        pl.BlockSpec((BLK, 1), lambda i: (i, 0)),
        pl.BlockSpec((1, D), lambda i: (0, 0)),
    ],
    out_specs=pl.BlockSpec((BLK, D), lambda i: (i, 0)),
    out_shape=jax.ShapeDtypeStruct((N_NODES, D), jnp.float32),
)


def kernel(x, edge_index, W1, b1, W2, b2):
    row = edge_index[0].astype(jnp.int32)
    col = edge_index[1].astype(jnp.int32)
    row2 = row.reshape(NW * NCHUNKS, CHUNK)
    col2 = col.reshape(NW * NCHUNKS, CHUNK)
    row2a = row.reshape(NW * ANCH, ACHUNK)
    col2a = col.reshape(NW * ANCH, ACHUNK)

    degp = _deg_call(col2)
    d0 = degp[0, :N_NODES, None]
    d1 = degp[1, :N_NODES, None]

    dis, g1 = _tc_a_call(d0, d1, x, W1)
    acc1 = _agg_call(g1, row2a, col2a)
    g2 = _tc_b_call(acc1[0, :N_NODES], acc1[1, :N_NODES], g1, dis,
                    b1.reshape(1, D), W2)
    acc2 = _agg_call(g2, row2a, col2a)
    out = _tc_c_call(acc2[0, :N_NODES], acc2[1, :N_NODES], g2, dis,
                     b2.reshape(1, D))
    return out
